# Initial kernel scaffold; baseline (speedup 1.0000x reference)
import jax
import jax.numpy as jnp
from jax.experimental import pallas as pl


def kernel(x, Ws, bs, Wf, bf, Wo, bo):
    raise NotImplementedError("write your pallas kernel here")


# speedup vs baseline: 5.4795x; 5.4795x over previous
import functools

import jax
import jax.numpy as jnp
from jax.experimental import pallas as pl
from jax.experimental.pallas import tpu as pltpu

RV = 256
BIG = 1e30


def _gravnet_block(x_all_ref, ws_ref, bs_ref, wf_ref, bf_ref, wo_ref, bo_ref,
                   out_ref, *, n_neigh, v_total):
    r = pl.program_id(1)
    x_all = x_all_ref[0]

    f_all = jnp.dot(x_all, wf_ref[...],
                    preferred_element_type=jnp.float32) + bf_ref[...]

    c_all_t = jax.lax.dot_general(
        ws_ref[...], x_all, (((0,), (1,)), ((), ())),
        preferred_element_type=jnp.float32) + bs_ref[...][:, None]

    r0 = r * RV
    x_row = x_all_ref[0, pl.ds(r0, RV), :]
    c_row = jnp.dot(x_row, ws_ref[...],
                    preferred_element_type=jnp.float32) + bs_ref[...]

    n_dim = c_row.shape[1]
    dist = jnp.zeros((RV, v_total), dtype=jnp.float32)
    for d in range(n_dim):
        t = c_row[:, d:d + 1] - c_all_t[d:d + 1, :]
        dist = dist + t * t

    col_ids = jax.lax.broadcasted_iota(jnp.int32, (RV, v_total), 1)
    row_ids = jax.lax.broadcasted_iota(jnp.int32, (RV, v_total), 0) + r0
    dist = jnp.where(col_ids == row_ids, BIG, dist)

    k = n_neigh - 1
    n_prop = f_all.shape[1]

    def body(_, carry):
        d, mx, sm = carry
        m = jnp.min(d, axis=1)
        eq = d == m[:, None]
        idx = jnp.min(jnp.where(eq, col_ids, v_total), axis=1)
        onehot = (col_ids == idx[:, None]).astype(jnp.float32)
        d = jnp.where(col_ids == idx[:, None], BIG, d)
        feat = jnp.dot(onehot, f_all, preferred_element_type=jnp.float32)
        wf = jnp.exp(-m)[:, None] * feat
        return d, jnp.maximum(mx, wf), sm + wf

    mx0 = jnp.full((RV, n_prop), -BIG, dtype=jnp.float32)
    sm0 = jnp.zeros((RV, n_prop), dtype=jnp.float32)
    _, mx, sm = jax.lax.fori_loop(0, k, body, (dist, mx0, sm0))

    cat = jnp.concatenate([x_row, mx, sm * (1.0 / k)], axis=1)
    out = jnp.dot(cat, wo_ref[...], preferred_element_type=jnp.float32)
    out_ref[0] = jnp.tanh(out + bo_ref[...])


def kernel(x, Ws, bs, Wf, bf, Wo, bo):
    b, v, f_in = x.shape
    n_neigh = 40
    n_filters = Wo.shape[1]
    grid = (b, v // RV)

    body = functools.partial(_gravnet_block, n_neigh=n_neigh, v_total=v)
    return pl.pallas_call(
        body,
        grid=grid,
        in_specs=[
            pl.BlockSpec((1, v, f_in), lambda bi, ri: (bi, 0, 0)),
            pl.BlockSpec(Ws.shape, lambda bi, ri: (0, 0)),
            pl.BlockSpec(bs.shape, lambda bi, ri: (0,)),
            pl.BlockSpec(Wf.shape, lambda bi, ri: (0, 0)),
            pl.BlockSpec(bf.shape, lambda bi, ri: (0,)),
            pl.BlockSpec(Wo.shape, lambda bi, ri: (0, 0)),
            pl.BlockSpec(bo.shape, lambda bi, ri: (0,)),
        ],
        out_specs=pl.BlockSpec((1, RV, n_filters), lambda bi, ri: (bi, ri, 0)),
        out_shape=jax.ShapeDtypeStruct((b, v, n_filters), jnp.float32),
    )(x, Ws, bs, Wf, bf, Wo, bo)
